# baseline (speedup 1.0000x reference)
import functools

import jax
import jax.numpy as jnp
from jax import lax
from jax.experimental import pallas as pl
from jax.experimental.pallas import tpu as pltpu
from jax.experimental.pallas import tpu_sc as plsc

V = 100000
E = 128
B = 1024
H = 50

NC = 2
NS = 16
NW = NC * NS
ROWS_PER_W = B // NW
CHUNK_ROWS = 2
CHUNK_IDX = CHUNK_ROWS * H
NCHUNK = ROWS_PER_W // CHUNK_ROWS
LANES = 16
NREG = E // LANES


def _sc_gather_mean(x_r, emb):
    mesh = plsc.VectorSubcoreMesh(core_axis_name="c", subcore_axis_name="s")

    @functools.partial(
        pl.kernel,
        mesh=mesh,
        out_type=jax.ShapeDtypeStruct((B, E), jnp.float32),
        scratch_types=[
            pltpu.VMEM((NCHUNK, CHUNK_IDX), jnp.int32),
            pltpu.VMEM((CHUNK_IDX, E), jnp.float32),
            pltpu.VMEM((ROWS_PER_W, E), jnp.float32),
            pltpu.SemaphoreType.DMA,
        ],
    )
    def k(x_hbm, emb_hbm, out_hbm, idx_v, buf_v, acc_v, sem):
        wid = lax.axis_index("s") * NC + lax.axis_index("c")
        pltpu.sync_copy(x_hbm.at[wid], idx_v)

        def chunk_body(c, carry):
            pltpu.async_copy(emb_hbm.at[idx_v.at[c]], buf_v, sem).wait()

            def j_body(j, accs):
                return tuple(
                    accs[r * NREG + kk]
                    + buf_v[r * H + j, pl.ds(kk * LANES, LANES)]
                    for r in range(CHUNK_ROWS)
                    for kk in range(NREG)
                )

            init = tuple(
                jnp.zeros((LANES,), jnp.float32)
                for _ in range(CHUNK_ROWS * NREG)
            )
            accs = lax.fori_loop(0, H, j_body, init)
            scale = jnp.float32(1.0 / H)
            for r in range(CHUNK_ROWS):
                for kk in range(NREG):
                    acc_v[c * CHUNK_ROWS + r, pl.ds(kk * LANES, LANES)] = (
                        accs[r * NREG + kk] * scale
                    )
            return carry

        lax.fori_loop(0, NCHUNK, chunk_body, 0)
        pltpu.sync_copy(acc_v, out_hbm.at[pl.ds(wid * ROWS_PER_W, ROWS_PER_W)])

    return k(x_r, emb)


VT = 2048
NV = (V + VT - 1) // VT
TAIL = V - (NV - 1) * VT
NBUF = 4


def _pa_body(avgb_ref, w_ref, b_ref, m_ref, macc):
    j = pl.program_id(0)
    l32 = jnp.dot(avgb_ref[...], w_ref[...],
                  preferred_element_type=jnp.float32)
    l = (l32 + b_ref[...]).astype(jnp.bfloat16)

    @pl.when(j == 0)
    def _():
        macc[...] = l

    @pl.when((j > 0) & (j < NV - 1))
    def _():
        macc[...] = jnp.maximum(macc[...], l)

    @pl.when(j == NV - 1)
    def _():
        col = lax.broadcasted_iota(jnp.int32, (1, VT), 1)
        lm = jnp.where(col < TAIL, l, jnp.finfo(jnp.bfloat16).min)
        macc[...] = jnp.maximum(macc[...], lm)
        m_ref[...] = jnp.max(macc[...], axis=1, keepdims=True).astype(
            jnp.float32)


def _pb_body(avgb_ref, w_ref, b_ref, m_ref, s_ref, sacc):
    j = pl.program_id(0)
    l = jnp.dot(avgb_ref[...], w_ref[...],
                preferred_element_type=jnp.float32)
    e = jnp.exp(l + b_ref[...] - m_ref[...])

    @pl.when(j == 0)
    def _():
        sacc[...] = e

    @pl.when((j > 0) & (j < NV - 1))
    def _():
        sacc[...] = sacc[...] + e

    @pl.when(j == NV - 1)
    def _():
        col = lax.broadcasted_iota(jnp.int32, (1, VT), 1)
        sacc[...] = sacc[...] + jnp.where(col < TAIL, e, 0.0)
        s_ref[...] = jnp.sum(sacc[...], axis=1, keepdims=True)


NHALF = (NV - 1) // 2
HOFF = NHALF * VT


def _tc_stats(avgb, Wb, b2, interpret=False):
    m = pl.pallas_call(
        _pa_body,
        grid=(NV,),
        in_specs=[
            pl.BlockSpec((B, E), lambda j: (0, 0)),
            pl.BlockSpec((E, VT), lambda j: (0, j)),
            pl.BlockSpec((1, VT), lambda j: (0, j)),
        ],
        out_specs=pl.BlockSpec((B, 1), lambda j: (0, 0)),
        out_shape=jax.ShapeDtypeStruct((B, 1), jnp.float32),
        scratch_shapes=[pltpu.VMEM((B, VT), jnp.bfloat16)],
        interpret=interpret,
    )(avgb, Wb, b2)

    s = pl.pallas_call(
        _pb_body,
        grid=(NV,),
        in_specs=[
            pl.BlockSpec((B, E), lambda j: (0, 0)),
            pl.BlockSpec((E, VT), lambda j: (0, j)),
            pl.BlockSpec((1, VT), lambda j: (0, j)),
            pl.BlockSpec((B, 1), lambda j: (0, 0)),
        ],
        out_specs=pl.BlockSpec((B, 1), lambda j: (0, 0)),
        out_shape=jax.ShapeDtypeStruct((B, 1), jnp.float32),
        scratch_shapes=[pltpu.VMEM((B, VT), jnp.float32)],
        interpret=interpret,
    )(avgb, Wb, b2, m)
    return m, s


def _pc_2out_body(avgb_ref, wa_ref, wb_ref, ba_ref, bb_ref, c_ref,
                  o1_ref, o2_ref):
    j = pl.program_id(0)
    la = jnp.dot(avgb_ref[...], wa_ref[...],
                 preferred_element_type=jnp.float32)
    o1_ref[...] = jnp.exp(la + ba_ref[...] - c_ref[...]).astype(jnp.bfloat16)

    @pl.when(j < NHALF)
    def _():
        lb = jnp.dot(avgb_ref[...], wb_ref[...],
                     preferred_element_type=jnp.float32)
        o2_ref[...] = jnp.exp(lb + bb_ref[...] - c_ref[...]).astype(
            jnp.bfloat16)


V1 = (NHALF + 1) * VT
V2 = V - V1


def _pc_write(avgb, Wb, b2, c, interpret=False):
    o1, o2 = pl.pallas_call(
        _pc_2out_body,
        grid=(NHALF + 1,),
        in_specs=[
            pl.BlockSpec((B, E), lambda j: (0, 0)),
            pl.BlockSpec((E, VT), lambda j: (0, j)),
            pl.BlockSpec(
                (E, VT),
                lambda j: (0, jnp.minimum(j, NHALF - 1) + NHALF + 1)),
            pl.BlockSpec((1, VT), lambda j: (0, j)),
            pl.BlockSpec(
                (1, VT),
                lambda j: (0, jnp.minimum(j, NHALF - 1) + NHALF + 1)),
            pl.BlockSpec((B, 1), lambda j: (0, 0)),
        ],
        out_specs=[
            pl.BlockSpec((B, VT), lambda j: (0, j)),
            pl.BlockSpec((B, VT), lambda j: (0, jnp.minimum(j, NHALF - 1))),
        ],
        out_shape=[
            jax.ShapeDtypeStruct((B, V1), jnp.bfloat16),
            jax.ShapeDtypeStruct((B, V2), jnp.bfloat16),
        ],
        compiler_params=pltpu.CompilerParams(
            dimension_semantics=("arbitrary",),
        ),
        interpret=interpret,
    )(avgb, Wb, Wb, b2, b2, c)
    return o1, o2


def kernel(x, emb, W, b):
    x_r = x.astype(jnp.int32).reshape(NW, NCHUNK, CHUNK_IDX)
    avg = _sc_gather_mean(x_r, emb)
    avgb = avg.astype(jnp.bfloat16)
    Wb = W.astype(jnp.bfloat16)
    b2 = b.reshape(1, V)
    m, s = _tc_stats(avgb, Wb, b2)
    c = m + jnp.log(s)
    o1, o2 = _pc_write(avgb, Wb, b2, c)
    return jnp.concatenate([o1, o2], axis=1).astype(jnp.float32)


# speedup vs baseline: 1.2815x; 1.2815x over previous
import functools

import jax
import jax.numpy as jnp
from jax import lax
from jax.experimental import pallas as pl
from jax.experimental.pallas import tpu as pltpu
from jax.experimental.pallas import tpu_sc as plsc

V = 100000
E = 128
B = 1024
H = 50

NC = 2
NS = 16
NW = NC * NS
ROWS_PER_W = B // NW
CHUNK_ROWS = 2
CHUNK_IDX = CHUNK_ROWS * H
NCHUNK = ROWS_PER_W // CHUNK_ROWS
LANES = 16
NREG = E // LANES


def _sc_gather_mean(x_r, emb):
    mesh = plsc.VectorSubcoreMesh(core_axis_name="c", subcore_axis_name="s")

    @functools.partial(
        pl.kernel,
        mesh=mesh,
        out_type=jax.ShapeDtypeStruct((B, E), jnp.float32),
        scratch_types=[
            pltpu.VMEM((NCHUNK, CHUNK_IDX), jnp.int32),
            pltpu.VMEM((CHUNK_IDX, E), jnp.float32),
            pltpu.VMEM((ROWS_PER_W, E), jnp.float32),
            pltpu.SemaphoreType.DMA,
        ],
    )
    def k(x_hbm, emb_hbm, out_hbm, idx_v, buf_v, acc_v, sem):
        wid = lax.axis_index("s") * NC + lax.axis_index("c")
        pltpu.sync_copy(x_hbm.at[wid], idx_v)

        def chunk_body(c, carry):
            pltpu.async_copy(emb_hbm.at[idx_v.at[c]], buf_v, sem).wait()

            def j_body(j, accs):
                return tuple(
                    accs[r * NREG + kk]
                    + buf_v[r * H + j, pl.ds(kk * LANES, LANES)]
                    for r in range(CHUNK_ROWS)
                    for kk in range(NREG)
                )

            init = tuple(
                jnp.zeros((LANES,), jnp.float32)
                for _ in range(CHUNK_ROWS * NREG)
            )
            accs = lax.fori_loop(0, H, j_body, init)
            scale = jnp.float32(1.0 / H)
            for r in range(CHUNK_ROWS):
                for kk in range(NREG):
                    acc_v[c * CHUNK_ROWS + r, pl.ds(kk * LANES, LANES)] = (
                        accs[r * NREG + kk] * scale
                    )
            return carry

        lax.fori_loop(0, NCHUNK, chunk_body, 0)
        pltpu.sync_copy(acc_v, out_hbm.at[pl.ds(wid * ROWS_PER_W, ROWS_PER_W)])

    return k(x_r, emb)


VT = 2048
NV = (V + VT - 1) // VT
TAIL = V - (NV - 1) * VT
VTS = 4096
NVS = (V + VTS - 1) // VTS
TAILS = V - (NVS - 1) * VTS


def _pa_body(avgb_ref, w_ref, b_ref, m_ref, macc):
    j = pl.program_id(0)
    l32 = jnp.dot(avgb_ref[...], w_ref[...],
                  preferred_element_type=jnp.float32)
    l = (l32 + b_ref[...]).astype(jnp.bfloat16)

    @pl.when(j == 0)
    def _():
        macc[...] = l

    @pl.when((j > 0) & (j < NVS - 1))
    def _():
        macc[...] = jnp.maximum(macc[...], l)

    @pl.when(j == NVS - 1)
    def _():
        col = lax.broadcasted_iota(jnp.int32, (1, VTS), 1)
        lm = jnp.where(col < TAILS, l, jnp.finfo(jnp.bfloat16).min)
        macc[...] = jnp.maximum(macc[...], lm)
        m_ref[...] = jnp.max(macc[...], axis=1, keepdims=True).astype(
            jnp.float32)


def _pb_body(avgb_ref, w_ref, b_ref, m_ref, s_ref, sacc):
    j = pl.program_id(0)
    l = jnp.dot(avgb_ref[...], w_ref[...],
                preferred_element_type=jnp.float32)
    e = jnp.exp(l + b_ref[...] - m_ref[...])

    @pl.when(j == 0)
    def _():
        sacc[...] = e

    @pl.when((j > 0) & (j < NVS - 1))
    def _():
        sacc[...] = sacc[...] + e

    @pl.when(j == NVS - 1)
    def _():
        col = lax.broadcasted_iota(jnp.int32, (1, VTS), 1)
        sacc[...] = sacc[...] + jnp.where(col < TAILS, e, 0.0)
        s_ref[...] = jnp.sum(sacc[...], axis=1, keepdims=True)


def _tc_stats(avgb, Wb, b2, interpret=False):
    m = pl.pallas_call(
        _pa_body,
        grid=(NVS,),
        in_specs=[
            pl.BlockSpec((B, E), lambda j: (0, 0)),
            pl.BlockSpec((E, VTS), lambda j: (0, j)),
            pl.BlockSpec((1, VTS), lambda j: (0, j)),
        ],
        out_specs=pl.BlockSpec((B, 1), lambda j: (0, 0)),
        out_shape=jax.ShapeDtypeStruct((B, 1), jnp.float32),
        scratch_shapes=[pltpu.VMEM((B, VTS), jnp.bfloat16)],
        interpret=interpret,
    )(avgb, Wb, b2)

    s = pl.pallas_call(
        _pb_body,
        grid=(NVS,),
        in_specs=[
            pl.BlockSpec((B, E), lambda j: (0, 0)),
            pl.BlockSpec((E, VTS), lambda j: (0, j)),
            pl.BlockSpec((1, VTS), lambda j: (0, j)),
            pl.BlockSpec((B, 1), lambda j: (0, 0)),
        ],
        out_specs=pl.BlockSpec((B, 1), lambda j: (0, 0)),
        out_shape=jax.ShapeDtypeStruct((B, 1), jnp.float32),
        scratch_shapes=[pltpu.VMEM((B, VTS), jnp.float32)],
        interpret=interpret,
    )(avgb, Wb, b2, m)
    return m, s


def _pcw_body(avgb_ref, w_ref, b_ref, c_ref, out_ref):
    l = jnp.dot(avgb_ref[...], w_ref[...],
                preferred_element_type=jnp.float32)
    out_ref[...] = jnp.exp(l + b_ref[...] - c_ref[...]).astype(jnp.bfloat16)


def _pc_write(avgb, Wb, b2, c, interpret=False):
    return pl.pallas_call(
        _pcw_body,
        grid=(NV,),
        in_specs=[
            pl.BlockSpec((B, E), lambda j: (0, 0)),
            pl.BlockSpec((E, VT), lambda j: (0, j)),
            pl.BlockSpec((1, VT), lambda j: (0, j)),
            pl.BlockSpec((B, 1), lambda j: (0, 0)),
        ],
        out_specs=pl.BlockSpec((B, VT), lambda j: (0, j)),
        out_shape=jax.ShapeDtypeStruct((B, V), jnp.bfloat16),
        compiler_params=pltpu.CompilerParams(
            dimension_semantics=("arbitrary",),
        ),
        interpret=interpret,
    )(avgb, Wb, b2, c)


def kernel(x, emb, W, b):
    x_r = x.astype(jnp.int32).reshape(NW, NCHUNK, CHUNK_IDX)
    avg = _sc_gather_mean(x_r, emb)
    avgb = avg.astype(jnp.bfloat16)
    Wb = W.astype(jnp.bfloat16)
    b2 = b.reshape(1, V)
    m, s = _tc_stats(avgb, Wb, b2)
    c = m + jnp.log(s)
    ob = _pc_write(avgb, Wb, b2, c)
    return ob.astype(jnp.float32)
